# Initial kernel scaffold; baseline (speedup 1.0000x reference)
import jax
import jax.numpy as jnp
from jax.experimental import pallas as pl


def kernel(inputs, features):
    raise NotImplementedError("write your pallas kernel here")


# speedup vs baseline: 3.2560x; 3.2560x over previous
import functools

import jax
import jax.numpy as jnp
from jax import lax
from jax.experimental import pallas as pl
from jax.experimental.pallas import tpu as pltpu
from jax.experimental.pallas import tpu_sc as plsc

D = 128
G = 128
NG = 4
CHUNK = G * NG


def _sc_gather(table, idx_groups):
    info = plsc.get_sparse_core_info()
    nc, ns = info.num_cores, info.num_subcores
    nw = nc * ns
    n_groups, _ = idx_groups.shape
    b = n_groups * G
    groups_per_w = n_groups // nw
    n_chunks = groups_per_w // NG
    mesh = plsc.VectorSubcoreMesh(core_axis_name="c", subcore_axis_name="s")

    @functools.partial(
        pl.kernel,
        mesh=mesh,
        out_type=jax.ShapeDtypeStruct((b, D), jnp.float32),
        scratch_types=[
            pltpu.VMEM((NG, G), jnp.int32),
            pltpu.VMEM((CHUNK, D), jnp.float32),
            pltpu.SemaphoreType.DMA,
        ],
    )
    def k(table_hbm, idx_hbm, out_hbm, idx_v, rows_v, sem):
        wid = lax.axis_index("s") * nc + lax.axis_index("c")
        g_base = wid * groups_per_w

        def body(i, carry):
            g0 = g_base + i * NG
            pltpu.sync_copy(idx_hbm.at[pl.ds(g0, NG)], idx_v)
            copies = [
                pltpu.async_copy(
                    table_hbm.at[idx_v.at[j]],
                    rows_v.at[pl.ds(j * G, G)],
                    sem,
                )
                for j in range(NG)
            ]
            for c in copies:
                c.wait()
            pltpu.sync_copy(rows_v, out_hbm.at[pl.ds(g0 * G, CHUNK)])
            return carry

        lax.fori_loop(0, n_chunks, body, 0)

    return k(table, idx_groups)


def kernel(inputs, features):
    batch, n_fields = inputs.shape
    idx_groups = inputs.reshape(-1, G)
    out = _sc_gather(features, idx_groups)
    return out.reshape(batch, n_fields, D)


# speedup vs baseline: 3.3861x; 1.0399x over previous
import functools

import jax
import jax.numpy as jnp
from jax import lax
from jax.experimental import pallas as pl
from jax.experimental.pallas import tpu as pltpu
from jax.experimental.pallas import tpu_sc as plsc

D = 128
G = 128
NG = 2
CHUNK = G * NG


def _sc_gather(table, idx_groups):
    info = plsc.get_sparse_core_info()
    nc, ns = info.num_cores, info.num_subcores
    nw = nc * ns
    n_groups, _ = idx_groups.shape
    b = n_groups * G
    groups_per_w = n_groups // nw
    n_chunks = groups_per_w // NG
    mesh = plsc.VectorSubcoreMesh(core_axis_name="c", subcore_axis_name="s")

    @functools.partial(
        pl.kernel,
        mesh=mesh,
        out_type=jax.ShapeDtypeStruct((b, D), jnp.float32),
        scratch_types=[
            pltpu.VMEM((groups_per_w, G), jnp.int32),
            pltpu.VMEM((CHUNK, D), jnp.float32),
            pltpu.VMEM((CHUNK, D), jnp.float32),
            pltpu.SemaphoreType.DMA,
            pltpu.SemaphoreType.DMA,
            pltpu.SemaphoreType.DMA,
            pltpu.SemaphoreType.DMA,
        ],
    )
    def k(table_hbm, idx_hbm, out_hbm, idx_v, rows0, rows1, g0, g1, o0, o1):
        wid = lax.axis_index("s") * nc + lax.axis_index("c")
        g_base = wid * groups_per_w
        rows = (rows0, rows1)
        gsem = (g0, g1)
        osem = (o0, o1)

        pltpu.sync_copy(idx_hbm.at[pl.ds(g_base, groups_per_w)], idx_v)

        def fire(c, p):
            for j in range(NG):
                pltpu.async_copy(
                    table_hbm.at[idx_v.at[c * NG + j]],
                    rows[p].at[pl.ds(j * G, G)],
                    gsem[p],
                )

        def wait_gather(p):
            for j in range(NG):
                pltpu.make_async_copy(
                    table_hbm.at[idx_v.at[j]],
                    rows[p].at[pl.ds(j * G, G)],
                    gsem[p],
                ).wait()

        def out_start(c, p):
            pltpu.async_copy(
                rows[p], out_hbm.at[pl.ds((g_base + c * NG) * G, CHUNK)], osem[p]
            )

        def wait_out(p):
            pltpu.make_async_copy(
                rows[p], out_hbm.at[pl.ds(g_base * G, CHUNK)], osem[p]
            ).wait()

        fire(0, 0)
        fire(1, 1)
        wait_gather(0)
        out_start(0, 0)

        def body(t, carry):
            c = 2 * t + 1
            wait_out(0)
            fire(c + 1, 0)
            wait_gather(1)
            out_start(c, 1)

            wait_out(1)
            fire(c + 2, 1)
            wait_gather(0)
            out_start(c + 1, 0)
            return carry

        lax.fori_loop(0, (n_chunks - 2) // 2, body, 0)

        wait_out(0)
        wait_gather(1)
        out_start(n_chunks - 1, 1)
        wait_out(1)

    return k(table, idx_groups)


def kernel(inputs, features):
    batch, n_fields = inputs.shape
    idx_groups = inputs.reshape(-1, G)
    out = _sc_gather(features, idx_groups)
    return out.reshape(batch, n_fields, D)


# speedup vs baseline: 3.3876x; 1.0004x over previous
import functools

import jax
import jax.numpy as jnp
from jax import lax
from jax.experimental import pallas as pl
from jax.experimental.pallas import tpu as pltpu
from jax.experimental.pallas import tpu_sc as plsc

D = 128
G = 128
NB = 4


def _sc_gather(table, idx_groups):
    info = plsc.get_sparse_core_info()
    nc, ns = info.num_cores, info.num_subcores
    nw = nc * ns
    n_groups, _ = idx_groups.shape
    b = n_groups * G
    gpw = n_groups // nw
    mesh = plsc.VectorSubcoreMesh(core_axis_name="c", subcore_axis_name="s")

    @functools.partial(
        pl.kernel,
        mesh=mesh,
        out_type=jax.ShapeDtypeStruct((b, D), jnp.float32),
        scratch_types=[
            pltpu.VMEM((gpw, G), jnp.int32),
        ]
        + [pltpu.VMEM((G, D), jnp.float32) for _ in range(NB)]
        + [pltpu.SemaphoreType.DMA for _ in range(2 * NB)],
    )
    def k(table_hbm, idx_hbm, out_hbm, idx_v, *bufs_sems):
        rows = bufs_sems[:NB]
        gsem = bufs_sems[NB : 2 * NB]
        osem = bufs_sems[2 * NB :]
        wid = lax.axis_index("s") * nc + lax.axis_index("c")
        g_base = wid * gpw

        pltpu.sync_copy(idx_hbm.at[pl.ds(g_base, gpw)], idx_v)

        def fire(c, p):
            pltpu.async_copy(table_hbm.at[idx_v.at[c]], rows[p], gsem[p])

        def wait_gather(p):
            pltpu.make_async_copy(table_hbm.at[idx_v.at[0]], rows[p], gsem[p]).wait()

        def out_start(c, p):
            pltpu.async_copy(rows[p], out_hbm.at[pl.ds((g_base + c) * G, G)], osem[p])

        def wait_out(p):
            pltpu.make_async_copy(
                rows[p], out_hbm.at[pl.ds(g_base * G, G)], osem[p]
            ).wait()

        for p in range(NB):
            fire(p, p)
        wait_gather(0)
        out_start(0, 0)

        n_steady = gpw - NB
        assert (n_steady - 1 + 1) % NB == 0

        def body(t, carry):
            for q in range(NB):
                c = NB * t + 1 + q
                pb = q % NB
                cb = (q + 1) % NB
                wait_out(pb)
                fire_c = c + NB - 1
                pltpu.async_copy(
                    table_hbm.at[idx_v.at[fire_c]], rows[pb], gsem[pb]
                )
                wait_gather(cb)
                out_start(c, cb)
            return carry

        lax.fori_loop(0, n_steady // NB, body, 0)

        for c in range(gpw - NB + 1, gpw):
            pb = (c - 1) % NB
            cb = c % NB
            wait_out(pb)
            wait_gather(cb)
            out_start(c, cb)
        wait_out((gpw - 1) % NB)

    return k(table, idx_groups)


def kernel(inputs, features):
    batch, n_fields = inputs.shape
    idx_groups = inputs.reshape(-1, G)
    out = _sc_gather(features, idx_groups)
    return out.reshape(batch, n_fields, D)


# speedup vs baseline: 5.7065x; 1.6845x over previous
import functools

import jax
import jax.numpy as jnp
from jax import lax
from jax.experimental import pallas as pl
from jax.experimental.pallas import tpu as pltpu
from jax.experimental.pallas import tpu_sc as plsc

D = 128
F = 26
SPC = 4
G = F * SPC
NB = 4


def _sc_gather(table, idx_groups, batch):
    info = plsc.get_sparse_core_info()
    nc, ns = info.num_cores, info.num_subcores
    nw = nc * ns
    mesh = plsc.VectorSubcoreMesh(core_axis_name="c", subcore_axis_name="s")
    cpw = (batch // SPC) // nw
    spw = batch // nw

    @functools.partial(
        pl.kernel,
        mesh=mesh,
        out_type=jax.ShapeDtypeStruct((batch, F, D), jnp.float32),
        scratch_types=[
            pltpu.VMEM((cpw, G), jnp.int32),
        ]
        + [pltpu.VMEM((G, D), jnp.float32) for _ in range(NB)]
        + [pltpu.SemaphoreType.DMA for _ in range(2 * NB)],
    )
    def k(table_hbm, idx_hbm, out_hbm, idx_v, *bufs_sems):
        rows = bufs_sems[:NB]
        gsem = bufs_sems[NB : 2 * NB]
        osem = bufs_sems[2 * NB :]
        wid = lax.axis_index("s") * nc + lax.axis_index("c")
        c_base = wid * cpw
        s_base = wid * spw

        pltpu.sync_copy(idx_hbm.at[pl.ds(c_base, cpw)], idx_v)

        def fire(c, p):
            pltpu.async_copy(table_hbm.at[idx_v.at[c]], rows[p], gsem[p])

        def wait_gather(p):
            pltpu.make_async_copy(table_hbm.at[idx_v.at[0]], rows[p], gsem[p]).wait()

        def out_start(c, p):
            for m in range(SPC):
                pltpu.async_copy(
                    rows[p].at[pl.ds(m * F, F)],
                    out_hbm.at[s_base + c * SPC + m],
                    osem[p],
                )

        def wait_out(p):
            for m in range(SPC):
                pltpu.make_async_copy(
                    rows[p].at[pl.ds(m * F, F)], out_hbm.at[s_base], osem[p]
                ).wait()

        for p in range(NB):
            fire(p, p)
        wait_gather(0)
        out_start(0, 0)

        n_steady = cpw - NB
        assert n_steady % NB == 0

        def body(t, carry):
            for q in range(NB):
                c = NB * t + 1 + q
                pb = q % NB
                cb = (q + 1) % NB
                wait_out(pb)
                fire(c + NB - 1, pb)
                wait_gather(cb)
                out_start(c, cb)
            return carry

        lax.fori_loop(0, n_steady // NB, body, 0)

        for c in range(cpw - NB + 1, cpw):
            wait_out((c - 1) % NB)
            wait_gather(c % NB)
            out_start(c, c % NB)
        wait_out((cpw - 1) % NB)

    return k(table, idx_groups)


def kernel(inputs, features):
    batch, n_fields = inputs.shape
    idx_groups = inputs.reshape(-1, G)
    return _sc_gather(features, idx_groups, batch)
